# Initial kernel scaffold; baseline (speedup 1.0000x reference)
import jax
import jax.numpy as jnp
from jax.experimental import pallas as pl


def kernel(inputs, table):
    raise NotImplementedError("write your pallas kernel here")


# speedup vs baseline: 1.9185x; 1.9185x over previous
import functools

import numpy as np
import jax
import jax.numpy as jnp
from jax import lax
from jax.experimental import pallas as pl
from jax.experimental.pallas import tpu as pltpu
from jax.experimental.pallas import tpu_sc as plsc

DIM = 64
_NC = 2
_NS = 16
_NW = _NC * _NS
_CH = 128


def _position_encoding(seq_len, d_model):
    positions = np.arange(seq_len)[:, np.newaxis]
    dims = np.arange(d_model)[np.newaxis, :]
    angles = positions / np.power(10000, 2 * (dims // 2) / d_model)
    pe = np.zeros(angles.shape, dtype=np.float32)
    pe[:, 0::2] = np.sin(angles[:, 0::2])
    pe[:, 1::2] = np.cos(angles[:, 1::2])
    return pe


@functools.lru_cache(maxsize=None)
def _make_kernel(B, L):
    N = B * L
    assert N % _NW == 0
    per_w = N // _NW
    assert per_w % _CH == 0
    n_ch = per_w // _CH
    assert (per_w % L) == 0

    mesh = plsc.VectorSubcoreMesh(core_axis_name="c", subcore_axis_name="s")

    @functools.partial(
        pl.kernel,
        mesh=mesh,
        compiler_params=pltpu.CompilerParams(use_tc_tiling_on_sc=False),
        out_type=jax.ShapeDtypeStruct((N, DIM), jnp.float32),
        scratch_types=[
            pltpu.VMEM((_CH,), jnp.int32),
            pltpu.VMEM((_CH, DIM), jnp.float32),
            pltpu.VMEM((L, DIM), jnp.float32),
            pltpu.SemaphoreType.DMA,
        ],
    )
    def k(idx_hbm, table_hbm, pe_hbm, out_hbm, idx_v, rows_v, pe_v, sem):
        wid = lax.axis_index("s") * _NC + lax.axis_index("c")
        pltpu.sync_copy(pe_hbm, pe_v)
        wbase = wid * per_w

        def chunk_body(c, carry):
            base = wbase + c * _CH
            pltpu.sync_copy(idx_hbm.at[pl.ds(base, _CH)], idx_v)
            pltpu.async_copy(table_hbm.at[idx_v], rows_v, sem).wait()

            cpos = lax.rem(c * _CH, L)

            def row_body(r, carry2):
                rp = lax.rem(cpos + r, L)
                for g in range(DIM // 16):
                    sl = pl.ds(g * 16, 16)
                    rows_v[r, sl] = rows_v[r, sl] + pe_v[rp, sl]
                return carry2

            lax.fori_loop(0, _CH, row_body, 0)

            pltpu.sync_copy(rows_v, out_hbm.at[pl.ds(base, _CH)])
            return carry

        lax.fori_loop(0, n_ch, chunk_body, 0)

    return k


def kernel(inputs, table):
    B, L = inputs.shape
    pe = jnp.asarray(_position_encoding(L, DIM))
    idx = inputs.reshape(-1)
    out = _make_kernel(B, L)(idx, table, pe)
    return out.reshape(B, L, DIM)


# speedup vs baseline: 2.3315x; 1.2153x over previous
import functools

import numpy as np
import jax
import jax.numpy as jnp
from jax import lax
from jax.experimental import pallas as pl
from jax.experimental.pallas import tpu as pltpu
from jax.experimental.pallas import tpu_sc as plsc

DIM = 64
_NC = 2
_NS = 16
_NW = _NC * _NS
_CH = 128


def _position_encoding(seq_len, d_model):
    positions = np.arange(seq_len)[:, np.newaxis]
    dims = np.arange(d_model)[np.newaxis, :]
    angles = positions / np.power(10000, 2 * (dims // 2) / d_model)
    pe = np.zeros(angles.shape, dtype=np.float32)
    pe[:, 0::2] = np.sin(angles[:, 0::2])
    pe[:, 1::2] = np.cos(angles[:, 1::2])
    return pe


@functools.lru_cache(maxsize=None)
def _make_kernel(B, L):
    N = B * L
    assert N % _NW == 0
    per_w = N // _NW
    assert per_w % (2 * _CH) == 0
    n_ch = per_w // _CH
    assert (per_w % L) == 0

    mesh = plsc.VectorSubcoreMesh(core_axis_name="c", subcore_axis_name="s")

    @functools.partial(
        pl.kernel,
        mesh=mesh,
        compiler_params=pltpu.CompilerParams(use_tc_tiling_on_sc=False),
        out_type=jax.ShapeDtypeStruct((N, DIM), jnp.float32),
        scratch_types=[
            pltpu.VMEM((per_w,), jnp.int32),
            pltpu.VMEM((L, DIM), jnp.float32),
            pltpu.VMEM((_CH, DIM), jnp.float32),
            pltpu.VMEM((_CH, DIM), jnp.float32),
            pltpu.VMEM((_CH, DIM), jnp.float32),
            pltpu.VMEM((_CH, DIM), jnp.float32),
            pltpu.SemaphoreType.DMA,
            pltpu.SemaphoreType.DMA,
            pltpu.SemaphoreType.DMA,
            pltpu.SemaphoreType.DMA,
        ],
    )
    def k(idx_hbm, table_hbm, pe_hbm, out_hbm,
          idx_v, pe_v, g0, g1, o0, o1, sg0, sg1, so0, so1):
        wid = lax.axis_index("s") * _NC + lax.axis_index("c")
        wbase = wid * per_w
        pltpu.sync_copy(idx_hbm.at[pl.ds(wbase, per_w)], idx_v)
        pltpu.sync_copy(pe_hbm, pe_v)

        gbuf = (g0, g1)
        obuf = (o0, o1)
        gsem = (sg0, sg1)
        osem = (so0, so1)

        def gather_desc(c, b):
            return pltpu.make_async_copy(
                table_hbm.at[idx_v.at[pl.ds(c * _CH, _CH)]], gbuf[b], gsem[b])

        def scatter_desc(c, b):
            return pltpu.make_async_copy(
                obuf[b], out_hbm.at[pl.ds(wbase + c * _CH, _CH)], osem[b])

        gather_desc(0, 0).start()
        gather_desc(1, 1).start()

        def step(cc, carry):
            for b in range(2):
                c = 2 * cc + b
                gather_desc(c, b).wait()

                @pl.when(c >= 2)
                def _():
                    scatter_desc(c, b).wait()

                cpos = lax.rem(c * _CH, L)
                gb, ob = gbuf[b], obuf[b]

                def row_body(r, rp):
                    for g in range(DIM // 16):
                        sl = pl.ds(g * 16, 16)
                        ob[r, sl] = gb[r, sl] + pe_v[rp, sl]
                    rp = rp + 1
                    return jnp.where(rp == L, 0, rp)

                lax.fori_loop(0, _CH, row_body, cpos, unroll=2)

                scatter_desc(c, b).start()

                @pl.when(c + 2 < n_ch)
                def _():
                    gather_desc(c + 2, b).start()
            return carry

        lax.fori_loop(0, n_ch // 2, step, 0)

        scatter_desc(n_ch - 2, 0).wait()
        scatter_desc(n_ch - 1, 1).wait()

    return k


def kernel(inputs, table):
    B, L = inputs.shape
    pe = jnp.asarray(_position_encoding(L, DIM))
    idx = inputs.reshape(-1)
    out = _make_kernel(B, L)(idx, table, pe)
    return out.reshape(B, L, DIM)
